# baseline (speedup 1.0000x reference)
import jax
import jax.numpy as jnp
from jax.experimental import pallas as pl
from jax.experimental.pallas import tpu as pltpu

H = 16
K = 4


def _knn_body(pw_ref, x_ref, pos_ref, wso_ref, bso_ref, waw_ref, baw_ref,
              wv_ref, bv_ref, p_ref, out_ref, v_scr):
    k = pl.program_id(1)
    pwr = jnp.maximum(pw_ref[0, 0], 0.0) + 1e-6
    xb = x_ref[0]
    p = p_ref[0]

    @pl.when(k == 0)
    def _():
        v_scr[...] = (jnp.dot(xb, wv_ref[0], preferred_element_type=jnp.float32)
                      + bv_ref[0])

    so = jnp.dot(xb, wso_ref[0], preferred_element_type=jnp.float32) + bso_ref[0]
    s = so + pos_ref[0]
    sx, sy = s[:, 0:1], s[:, 1:2]
    sp = jnp.dot(s, p[0:2, :], preferred_element_type=jnp.float32)
    d2 = ((sx * sx + sy * sy) + p[2:3, :]) - 2.0 * sp
    t4 = jnp.min(d2, axis=1, keepdims=True)
    for _ in range(3):
        t4 = jnp.min(jnp.where(d2 <= t4, jnp.inf, d2), axis=1, keepdims=True)
    dx = sx - p[0:1, :]
    dy = sy - p[1:2, :]
    dist = jnp.sqrt(dx * dx + dy * dy) + 1e-6
    w = jnp.where(d2 <= t4, jnp.exp(-pwr * dist), 0.0)
    aw = jnp.dot(xb, waw_ref[0], preferred_element_type=jnp.float32) + baw_ref[0]
    e = jnp.exp(aw - jnp.max(aw, axis=1, keepdims=True))
    lane = jax.lax.broadcasted_iota(jnp.int32, e.shape, 1)
    attn_col = (jnp.sum(jnp.where(lane == k, e, 0.0), axis=1, keepdims=True)
                / jnp.sum(e, axis=1, keepdims=True))
    w = w * (attn_col / jnp.sum(w, axis=1, keepdims=True))
    part = jnp.dot(w, v_scr[...], preferred_element_type=jnp.float32)

    @pl.when(k == 0)
    def _():
        out_ref[0, 0] = part

    @pl.when(k > 0)
    def _():
        out_ref[0, 0] += part


def _out_body(x_ref, w_ref, b_ref, o_ref):
    j = pl.program_id(1)
    part = jnp.dot(x_ref[0, 0], w_ref[0], preferred_element_type=jnp.float32)

    @pl.when(j == 0)
    def _():
        o_ref[0] = part + b_ref[...]

    @pl.when(j > 0)
    def _():
        o_ref[0] += part


def kernel(x, pos, W_so, b_so, W_aw, b_aw, W_v, b_v, W_o, b_o, shepard_power):
    b, n, d = x.shape
    c = d // H
    bh = b * H

    px, py = pos[..., 0], pos[..., 1]
    paug = jnp.stack([px, py, px * px + py * py], axis=1)
    wso4 = W_so.reshape(H * K, 2, d).transpose(0, 2, 1)
    bso4 = b_so.reshape(H * K, 1, 2)
    waw4 = W_aw.reshape(H, K, d).transpose(0, 2, 1)
    baw4 = b_aw.reshape(H, 1, K)
    wv4 = W_v.reshape(H, c, d).transpose(0, 2, 1)
    bv4 = b_v.reshape(H, 1, c)

    out_h = pl.pallas_call(
        _knn_body,
        grid=(bh, K),
        in_specs=[
            pl.BlockSpec(memory_space=pltpu.SMEM),
            pl.BlockSpec((1, n, d), lambda i, j: (i // H, 0, 0)),
            pl.BlockSpec((1, n, 2), lambda i, j: (i // H, 0, 0)),
            pl.BlockSpec((1, d, 2), lambda i, j: ((i % H) * K + j, 0, 0)),
            pl.BlockSpec((1, 1, 2), lambda i, j: ((i % H) * K + j, 0, 0)),
            pl.BlockSpec((1, d, K), lambda i, j: (i % H, 0, 0)),
            pl.BlockSpec((1, 1, K), lambda i, j: (i % H, 0, 0)),
            pl.BlockSpec((1, d, c), lambda i, j: (i % H, 0, 0)),
            pl.BlockSpec((1, 1, c), lambda i, j: (i % H, 0, 0)),
            pl.BlockSpec((1, 3, n), lambda i, j: (i // H, 0, 0)),
        ],
        out_specs=pl.BlockSpec((1, 1, n, c), lambda i, j: (i // H, i % H, 0, 0)),
        out_shape=jax.ShapeDtypeStruct((b, H, n, c), jnp.float32),
        scratch_shapes=[pltpu.VMEM((n, c), jnp.float32)],
    )(shepard_power.reshape(1, 1), x, pos, wso4, bso4, waw4, baw4,
      wv4, bv4, paug)

    out = pl.pallas_call(
        _out_body,
        grid=(b, H),
        in_specs=[
            pl.BlockSpec((1, 1, n, c), lambda i, j: (i, j, 0, 0)),
            pl.BlockSpec((1, c, d), lambda i, j: (j, 0, 0)),
            pl.BlockSpec((1, d), lambda i, j: (0, 0)),
        ],
        out_specs=pl.BlockSpec((1, n, d), lambda i, j: (i, 0, 0)),
        out_shape=jax.ShapeDtypeStruct((b, n, d), jnp.float32),
    )(out_h, W_o.T.reshape(H, c, d), b_o[None, :])
    return out


# speedup vs baseline: 1.2907x; 1.2907x over previous
import jax
import jax.numpy as jnp
from jax.experimental import pallas as pl
from jax.experimental.pallas import tpu as pltpu

H = 16
K = 4
NT = 256


def _proj_body(x_ref, pos_ref, wso_ref, bso_ref, waw_ref, baw_ref,
               wv_ref, bv_ref, samp_ref, attn_ref, val_ref):
    x = x_ref[0]
    so = jnp.dot(x, wso_ref[...], preferred_element_type=jnp.float32) + bso_ref[...]
    aw = jnp.dot(x, waw_ref[...], preferred_element_type=jnp.float32) + baw_ref[...]
    vv = jnp.dot(x, wv_ref[...], preferred_element_type=jnp.float32) + bv_ref[...]
    samp_ref[0] = so + jnp.tile(pos_ref[0], (1, so.shape[1] // 2))
    hk = aw.shape[1]
    gi = jax.lax.broadcasted_iota(jnp.int32, (hk, hk), 0)
    gj = jax.lax.broadcasted_iota(jnp.int32, (hk, hk), 1)
    g = ((gi // K) == (gj // K)).astype(jnp.float32)
    e = jnp.exp(aw - jnp.max(aw, axis=1, keepdims=True))
    attn_ref[0] = e / jnp.dot(e, g, preferred_element_type=jnp.float32)
    val_ref[0] = vv


def _knn_body(pw_ref, s_ref, a_ref, p_ref, v_ref, out_ref, v_scr):
    k = pl.program_id(1)
    c = v_ref.shape[-1]
    pwr = jnp.maximum(pw_ref[0, 0], 0.0) + 1e-6
    s = s_ref[0, 0]
    p = p_ref[0]
    sx, sy = s[:, 0:1], s[:, 1:2]

    @pl.when(k == 0)
    def _():
        v_scr[:, 0:c] = v_ref[0, 0]
        v_scr[:, c:c + 1] = jnp.ones_like(v_scr[:, c:c + 1])

    sp = jnp.dot(s, p[0:2, :], preferred_element_type=jnp.float32)
    d2 = ((sx * sx + sy * sy) + p[2:3, :]) - 2.0 * sp
    t4 = jnp.min(d2, axis=1, keepdims=True)
    for _ in range(3):
        t4 = jnp.min(jnp.where(d2 <= t4, jnp.inf, d2), axis=1, keepdims=True)
    dx = sx - p[0:1, :]
    dy = sy - p[1:2, :]
    dist = jnp.sqrt(dx * dx + dy * dy)
    w = jnp.where(d2 <= t4, jnp.exp(-pwr * dist), 0.0)
    part = jnp.dot(w, v_scr[...], preferred_element_type=jnp.float32)
    a = a_ref[0, 0]
    lane = jax.lax.broadcasted_iota(jnp.int32, a.shape, 1)
    attn_col = jnp.sum(jnp.where(lane == k, a, 0.0), axis=1, keepdims=True)
    contrib = part[:, 0:c] * (attn_col / part[:, c:c + 1])

    @pl.when(k == 0)
    def _():
        out_ref[0, 0] = contrib

    @pl.when(k > 0)
    def _():
        out_ref[0, 0] += contrib


def _out_body(x_ref, w_ref, b_ref, o_ref):
    j = pl.program_id(1)
    part = jnp.dot(x_ref[0, 0], w_ref[0], preferred_element_type=jnp.float32)

    @pl.when(j == 0)
    def _():
        o_ref[0] = part + b_ref[...]

    @pl.when(j > 0)
    def _():
        o_ref[0] += part


def kernel(x, pos, W_so, b_so, W_aw, b_aw, W_v, b_v, W_o, b_o, shepard_power):
    b, n, d = x.shape
    c = d // H
    bh = b * H

    samp, attn, vals = pl.pallas_call(
        _proj_body,
        grid=(b, n // NT),
        in_specs=[
            pl.BlockSpec((1, NT, d), lambda i, j: (i, j, 0)),
            pl.BlockSpec((1, NT, 2), lambda i, j: (i, j, 0)),
            pl.BlockSpec((d, H * K * 2), lambda i, j: (0, 0)),
            pl.BlockSpec((1, H * K * 2), lambda i, j: (0, 0)),
            pl.BlockSpec((d, H * K), lambda i, j: (0, 0)),
            pl.BlockSpec((1, H * K), lambda i, j: (0, 0)),
            pl.BlockSpec((d, d), lambda i, j: (0, 0)),
            pl.BlockSpec((1, d), lambda i, j: (0, 0)),
        ],
        out_specs=[
            pl.BlockSpec((1, NT, H * K * 2), lambda i, j: (i, j, 0)),
            pl.BlockSpec((1, NT, H * K), lambda i, j: (i, j, 0)),
            pl.BlockSpec((1, NT, d), lambda i, j: (i, j, 0)),
        ],
        out_shape=[
            jax.ShapeDtypeStruct((b, n, H * K * 2), jnp.float32),
            jax.ShapeDtypeStruct((b, n, H * K), jnp.float32),
            jax.ShapeDtypeStruct((b, n, d), jnp.float32),
        ],
    )(x, pos, W_so.T, b_so[None, :], W_aw.T, b_aw[None, :],
      W_v.T, b_v[None, :])

    px, py = pos[..., 0], pos[..., 1]
    paug = jnp.stack([px, py, px * px + py * py], axis=1)
    samp4 = samp.reshape(b, n, H * K, 2).transpose(0, 2, 1, 3)
    attn4 = attn.reshape(b, n, H, K).transpose(0, 2, 1, 3)
    vals4 = vals.reshape(b, n, H, c).transpose(0, 2, 1, 3)

    out_h = pl.pallas_call(
        _knn_body,
        grid=(bh, K),
        in_specs=[
            pl.BlockSpec(memory_space=pltpu.SMEM),
            pl.BlockSpec((1, 1, n, 2), lambda i, j: (i // H, (i % H) * K + j, 0, 0)),
            pl.BlockSpec((1, 1, n, K), lambda i, j: (i // H, i % H, 0, 0)),
            pl.BlockSpec((1, 3, n), lambda i, j: (i // H, 0, 0)),
            pl.BlockSpec((1, 1, n, c), lambda i, j: (i // H, i % H, 0, 0)),
        ],
        out_specs=pl.BlockSpec((1, 1, n, c), lambda i, j: (i // H, i % H, 0, 0)),
        out_shape=jax.ShapeDtypeStruct((b, H, n, c), jnp.float32),
        scratch_shapes=[pltpu.VMEM((n, c + 1), jnp.float32)],
    )(shepard_power.reshape(1, 1), samp4, attn4, paug, vals4)

    out = pl.pallas_call(
        _out_body,
        grid=(b, H),
        in_specs=[
            pl.BlockSpec((1, 1, n, c), lambda i, j: (i, j, 0, 0)),
            pl.BlockSpec((1, c, d), lambda i, j: (j, 0, 0)),
            pl.BlockSpec((1, d), lambda i, j: (0, 0)),
        ],
        out_specs=pl.BlockSpec((1, n, d), lambda i, j: (i, 0, 0)),
        out_shape=jax.ShapeDtypeStruct((b, n, d), jnp.float32),
    )(out_h, W_o.T.reshape(H, c, d), b_o[None, :])
    return out
